# baseline (speedup 1.0000x reference)
import functools

import jax
import jax.numpy as jnp
from jax import lax
from jax.experimental import pallas as pl
from jax.experimental.pallas import tpu as pltpu
from jax.experimental.pallas import tpu_sc as plsc

F32 = jnp.float32
I32 = jnp.int32
NCHUNK = 5
GCH = 40
SCH = 80


def _silu(x):
    return x / (1.0 + jnp.exp(-x))


def _dot(a, b):
    return jax.lax.dot_general(
        a, b, (((1,), (0,)), ((), ())), preferred_element_type=F32)


def _pack_bf16_pair(lo, hi):
    il = lax.bitcast_convert_type(lo, I32) + 0x8000
    ih = lax.bitcast_convert_type(hi, I32) + 0x8000
    return ((il >> 16) & 0xFFFF) | (ih & jnp.int32(-65536))


def _unpack_lo(w):
    return lax.bitcast_convert_type(w << 16, F32)


def _unpack_hi(w):
    return lax.bitcast_convert_type(w & jnp.int32(-65536), F32)


def _node_body(nf_ref, w_ref, sc_ref, xpk_ref, xdp_ref):
    y = _dot(nf_ref[...], w_ref[...]) * (1.0 / jnp.sqrt(128.0))
    sc_ref[...] = y[:, :128]
    dn = y[:, 256:384]
    xpk_ref[...] = _pack_bf16_pair(y[:, 128:256], dn)
    xdp_ref[...] = dn


def _node_linears(node_feats, w_cat, n, d):
    bn = 2000
    return pl.pallas_call(
        _node_body,
        grid=(n // bn,),
        in_specs=[
            pl.BlockSpec((bn, d), lambda i: (i, 0)),
            pl.BlockSpec((d, 3 * d), lambda i: (0, 0)),
        ],
        out_specs=[
            pl.BlockSpec((bn, d), lambda i: (i, 0)),
            pl.BlockSpec((bn, d), lambda i: (i, 0)),
            pl.BlockSpec((bn, d), lambda i: (i, 0)),
        ],
        out_shape=[
            jax.ShapeDtypeStruct((n, d), F32),
            jax.ShapeDtypeStruct((n, d), I32),
            jax.ShapeDtypeStruct((n, d), F32),
        ],
    )(node_feats, w_cat)


def _build_gather(ec, n, d):
    nc, ns = 2, 16
    nw = nc * ns
    per_w = ec // nw
    ch = GCH
    nch = per_w // ch

    mesh = plsc.VectorSubcoreMesh(
        core_axis_name="c", subcore_axis_name="s", num_cores=nc, num_subcores=ns)

    @functools.partial(
        pl.kernel,
        out_type=(
            jax.ShapeDtypeStruct((ec, d), I32),
            jax.ShapeDtypeStruct((ec, d), F32),
        ),
        mesh=mesh,
        scratch_types=[
            pltpu.VMEM((nch, ch), jnp.int32),
            pltpu.VMEM((nch, ch), jnp.int32),
            pltpu.VMEM((4, ch, d), I32),
            pltpu.VMEM((4, ch, d), F32),
            [pltpu.SemaphoreType.DMA] * 8,
            [pltpu.SemaphoreType.DMA] * 8,
        ],
    )
    def gather_k(xpk_hbm, xdp_hbm, snd_hbm, rcv_hbm,
                 gsnd_hbm, grdp_hbm,
                 idx_s, idx_r, bsnd, brcv, sg, sw):
        wid = lax.axis_index("s") * nc + lax.axis_index("c")
        base = wid * per_w
        c1 = pltpu.async_copy(snd_hbm.at[wid], idx_s, sg[0])
        c2 = pltpu.async_copy(rcv_hbm.at[wid], idx_r, sg[1])
        c1.wait()
        c2.wait()

        def issue_gathers(j, b):
            pltpu.async_copy(xpk_hbm.at[idx_s.at[j]], bsnd.at[b], sg[b])
            pltpu.async_copy(xdp_hbm.at[idx_r.at[j]], brcv.at[b], sg[4 + b])

        def wait_gathers(j, b):
            pltpu.make_async_copy(
                xpk_hbm.at[idx_s.at[j]], bsnd.at[b], sg[b]).wait()
            pltpu.make_async_copy(
                xdp_hbm.at[idx_r.at[j]], brcv.at[b], sg[4 + b]).wait()

        def issue_writes(j, b):
            e0 = base + j * ch
            pltpu.async_copy(bsnd.at[b], gsnd_hbm.at[pl.ds(e0, ch)], sw[b])
            pltpu.async_copy(brcv.at[b], grdp_hbm.at[pl.ds(e0, ch)], sw[4 + b])

        def wait_writes(j, b):
            e0 = base + j * ch
            pltpu.make_async_copy(
                bsnd.at[b], gsnd_hbm.at[pl.ds(e0, ch)], sw[b]).wait()
            pltpu.make_async_copy(
                brcv.at[b], grdp_hbm.at[pl.ds(e0, ch)], sw[4 + b]).wait()

        issue_gathers(0, 0)
        issue_gathers(1, 1)
        nq = (nch - 2) // 4

        def outer(q, carry):
            for b in range(4):
                j = 4 * q + b
                wait_gathers(j, b)
                issue_writes(j, b)

                @pl.when(j >= 2)
                def _():
                    wait_writes(j - 2, (b + 2) % 4)

                issue_gathers(j + 2, (b + 2) % 4)
            return carry

        lax.fori_loop(0, nq, outer, 0)
        if nq > 0:
            wait_writes(4 * nq - 2, (4 * nq - 2) % 4)
            wait_writes(4 * nq - 1, (4 * nq - 1) % 4)
        for j in range(4 * nq, nch):
            wait_gathers(j, j % 4)
            issue_writes(j, j % 4)
            wait_writes(j, j % 4)
            if j + 2 < nch:
                issue_gathers(j + 2, (j + 2) % 4)

    return gather_k


def _mlp_body(ef_ref, ea_ref, gsnd_ref, grdp_ref,
              w1_ref, w2_ref, w3_ref, w4_ref, out_ref):
    w = gsnd_ref[...]
    gup = _unpack_lo(w)
    gds = _unpack_hi(w)[:, :64]
    aug = jnp.concatenate(
        [ef_ref[...], gds, grdp_ref[...][:, :64]], axis=1)
    h = _silu(_dot(aug, w1_ref[...]) * (1.0 / jnp.sqrt(136.0)))
    h = _silu(_dot(h, w2_ref[...]) * 0.0625)
    h = _silu(_dot(h, w3_ref[...]) * 0.0625)
    tpw = _dot(h, w4_ref[...]) * 0.0625
    ea = ea_ref[...]
    out_ref[0] = tpw[:, :128] * gup * ea[:, 0:1]
    wx = tpw[:, 128:] * gup
    out_ref[1] = wx * ea[:, 1:2]
    out_ref[2] = wx * ea[:, 2:3]
    out_ref[3] = wx * ea[:, 3:4]


def _edge_mlp(edge_feats, edge_attrs, gsnd, grdp, w1, w2, w3, w4, ec, d):
    be = 2000
    return pl.pallas_call(
        _mlp_body,
        grid=(ec // be,),
        in_specs=[
            pl.BlockSpec((be, 8), lambda i: (i, 0)),
            pl.BlockSpec((be, 4), lambda i: (i, 0)),
            pl.BlockSpec((be, d), lambda i: (i, 0)),
            pl.BlockSpec((be, d), lambda i: (i, 0)),
            pl.BlockSpec((136, 256), lambda i: (0, 0)),
            pl.BlockSpec((256, 256), lambda i: (0, 0)),
            pl.BlockSpec((256, 256), lambda i: (0, 0)),
            pl.BlockSpec((256, 256), lambda i: (0, 0)),
        ],
        out_specs=pl.BlockSpec((4, be, d), lambda i: (0, i, 0)),
        out_shape=jax.ShapeDtypeStruct((4, ec, d), F32),
    )(edge_feats, edge_attrs, gsnd, grdp, w1, w2, w3, w4)


def _build_scatter(ec, n, d):
    nc, ns = 2, 16
    per_t = ec // ns
    ch = SCH
    nch = per_t // ch
    nb = (n // ns) // 8 * 8
    tail = n - ns * nb

    mesh = plsc.VectorSubcoreMesh(
        core_axis_name="c", subcore_axis_name="s", num_cores=nc, num_subcores=ns)

    @functools.partial(
        pl.kernel,
        out_type=jax.ShapeDtypeStruct((4, n, d), F32),
        mesh=mesh,
        scratch_types=[
            pltpu.VMEM_SHARED((n, d), F32),
            pltpu.VMEM((nch, ch), jnp.int32),
            pltpu.VMEM((4, ch, d), F32),
            [pltpu.SemaphoreType.DMA] * 4,
            [pltpu.SemaphoreType.DMA] * 4,
        ],
    )
    def scatter_k(mji_hbm, rcv_hbm, zeros_hbm, msg_hbm,
                  acc, idxall, mbuf, sg, sa):
        c = lax.axis_index("c")
        s = lax.axis_index("s")
        pltpu.async_copy(rcv_hbm.at[s], idxall, sg[0]).wait()
        for p in range(2):
            g = c * 2 + p
            pltpu.sync_copy(zeros_hbm, acc.at[pl.ds(s * nb, nb)])

            @pl.when(s == ns - 1)
            def _zero_tail():
                pltpu.sync_copy(zeros_hbm.at[pl.ds(0, tail)],
                                acc.at[pl.ds(ns * nb, tail)])

            plsc.subcore_barrier()

            def issue_fetch(j, b):
                e0 = s * per_t + j * ch
                pltpu.async_copy(mji_hbm.at[g, pl.ds(e0, ch)],
                                 mbuf.at[b], sg[b])

            def wait_fetch(j, b):
                e0 = s * per_t + j * ch
                pltpu.make_async_copy(mji_hbm.at[g, pl.ds(e0, ch)],
                                      mbuf.at[b], sg[b]).wait()

            def issue_add(j, b):
                pltpu.async_copy(mbuf.at[b], acc.at[idxall.at[j]],
                                 sa[b], add=True)

            def wait_add(j, b):
                pltpu.make_async_copy(mbuf.at[b], acc.at[idxall.at[j]],
                                      sa[b]).wait()

            issue_fetch(0, 0)
            issue_fetch(1, 1)
            nq = (nch - 2) // 4

            def outer(q, carry):
                for b in range(4):
                    j = 4 * q + b
                    wait_fetch(j, b)
                    issue_add(j, b)

                    @pl.when(j >= 2)
                    def _():
                        wait_add(j - 2, (b + 2) % 4)

                    issue_fetch(j + 2, (b + 2) % 4)
                return carry

            lax.fori_loop(0, nq, outer, 0)
            if nq > 0:
                wait_add(4 * nq - 2, (4 * nq - 2) % 4)
                wait_add(4 * nq - 1, (4 * nq - 1) % 4)
            for j in range(4 * nq, nch):
                wait_fetch(j, j % 4)
                issue_add(j, j % 4)
                wait_add(j, j % 4)
                if j + 2 < nch:
                    issue_fetch(j + 2, (j + 2) % 4)
            plsc.subcore_barrier()
            pltpu.sync_copy(acc.at[pl.ds(s * nb, nb)],
                            msg_hbm.at[g, pl.ds(s * nb, nb)])

            @pl.when(s == ns - 1)
            def _write_tail():
                pltpu.sync_copy(acc.at[pl.ds(ns * nb, tail)],
                                msg_hbm.at[g, pl.ds(ns * nb, tail)])

            plsc.subcore_barrier()

    return scatter_k


def _out_body(*refs):
    msg_refs = refs[:NCHUNK]
    w0_ref, w1_ref, out_ref = refs[NCHUNK:]
    m = msg_refs[0][...]
    for r in msg_refs[1:]:
        m = m + r[...]
    scale = 1.0 / (jnp.sqrt(128.0) * 32.0)
    out_ref[0] = _dot(m[0], w0_ref[...]) * scale
    out_ref[1] = _dot(m[1], w1_ref[...]) * scale
    out_ref[2] = _dot(m[2], w1_ref[...]) * scale
    out_ref[3] = _dot(m[3], w1_ref[...]) * scale


def _out_linears(msgs, w_lin0, w_lin1, n, d):
    bn = 1000
    return pl.pallas_call(
        _out_body,
        grid=(n // bn,),
        in_specs=[pl.BlockSpec((4, bn, d), lambda i: (0, i, 0))
                  for _ in range(NCHUNK)] + [
            pl.BlockSpec((d, d), lambda i: (0, 0)),
            pl.BlockSpec((d, d), lambda i: (0, 0)),
        ],
        out_specs=pl.BlockSpec((4, bn, d), lambda i: (0, i, 0)),
        out_shape=jax.ShapeDtypeStruct((4, n, d), F32),
    )(*msgs, w_lin0, w_lin1)


def kernel(node_attrs, node_feats, edge_attrs, edge_feats, edge_index,
           W_up, W_down, W_skip, W_mlp1, W_mlp2, W_mlp3, W_mlp4,
           W_lin0, W_lin1):
    n, d = node_feats.shape
    e = edge_index.shape[0]
    dd = W_down.shape[1]
    ec = e // NCHUNK

    sender = edge_index[:, 0]
    receiver = edge_index[:, 1]
    w_cat = jnp.concatenate(
        [W_skip, W_up, W_down, jnp.zeros((d, d - dd), F32)], axis=1)

    sc, xpk, xdp = _node_linears(node_feats, w_cat, n, d)

    gather_k = _build_gather(ec, n, d)
    scatter_k = _build_scatter(ec, n, d)
    zeros = jnp.zeros(((n // 16) // 8 * 8, d), F32)

    gathered = []
    for i in range(NCHUNK):
        sl = slice(i * ec, (i + 1) * ec)
        snd3 = sender[sl].reshape(32, -1, GCH)
        rcv3g = receiver[sl].reshape(32, -1, GCH)
        gathered.append(gather_k(xpk, xdp, snd3, rcv3g))

    msgs = []
    for i in range(NCHUNK):
        sl = slice(i * ec, (i + 1) * ec)
        gsnd, grdp = gathered[i]
        mji = _edge_mlp(edge_feats[sl], edge_attrs[sl], gsnd, grdp,
                        W_mlp1, W_mlp2, W_mlp3, W_mlp4, ec, d)
        rcv3s = receiver[sl].reshape(16, -1, SCH)
        msgs.append(scatter_k(mji, rcv3s, zeros))

    out4 = _out_linears(msgs, W_lin0, W_lin1, n, d)
    reshaped = jnp.transpose(out4, (1, 2, 0))
    return (reshaped, sc)


# speedup vs baseline: 1.0251x; 1.0251x over previous
import functools

import jax
import jax.numpy as jnp
from jax import lax
from jax.experimental import pallas as pl
from jax.experimental.pallas import tpu as pltpu
from jax.experimental.pallas import tpu_sc as plsc

F32 = jnp.float32
I32 = jnp.int32
NCHUNK = 5
GCH = 80
SCH = 80


def _silu(x):
    return x / (1.0 + jnp.exp(-x))


def _dot(a, b):
    return jax.lax.dot_general(
        a, b, (((1,), (0,)), ((), ())), preferred_element_type=F32)


def _pack_bf16_pair(lo, hi):
    il = lax.bitcast_convert_type(lo, I32) + 0x8000
    ih = lax.bitcast_convert_type(hi, I32) + 0x8000
    return ((il >> 16) & 0xFFFF) | (ih & jnp.int32(-65536))


def _unpack_lo(w):
    return lax.bitcast_convert_type(w << 16, F32)


def _unpack_hi(w):
    return lax.bitcast_convert_type(w & jnp.int32(-65536), F32)


def _node_body(nf_ref, w_ref, sc_ref, xpk_ref, xdp_ref):
    y = _dot(nf_ref[...], w_ref[...]) * (1.0 / jnp.sqrt(128.0))
    sc_ref[...] = y[:, :128]
    dn = y[:, 256:384]
    xpk_ref[...] = _pack_bf16_pair(y[:, 128:256], dn)
    xdp_ref[...] = dn


def _node_linears(node_feats, w_cat, n, d):
    bn = 2000
    return pl.pallas_call(
        _node_body,
        grid=(n // bn,),
        in_specs=[
            pl.BlockSpec((bn, d), lambda i: (i, 0)),
            pl.BlockSpec((d, 3 * d), lambda i: (0, 0)),
        ],
        out_specs=[
            pl.BlockSpec((bn, d), lambda i: (i, 0)),
            pl.BlockSpec((bn, d), lambda i: (i, 0)),
            pl.BlockSpec((bn, d), lambda i: (i, 0)),
        ],
        out_shape=[
            jax.ShapeDtypeStruct((n, d), F32),
            jax.ShapeDtypeStruct((n, d), I32),
            jax.ShapeDtypeStruct((n, d), F32),
        ],
    )(node_feats, w_cat)


def _build_gather(ec, n, d):
    nc, ns = 2, 16
    nw = nc * ns
    per_w = ec // nw
    ch = GCH
    nch = per_w // ch

    mesh = plsc.VectorSubcoreMesh(
        core_axis_name="c", subcore_axis_name="s", num_cores=nc, num_subcores=ns)

    @functools.partial(
        pl.kernel,
        out_type=(
            jax.ShapeDtypeStruct((ec, d), I32),
            jax.ShapeDtypeStruct((ec, d), F32),
        ),
        mesh=mesh,
        scratch_types=[
            pltpu.VMEM((nch, ch), jnp.int32),
            pltpu.VMEM((nch, ch), jnp.int32),
            pltpu.VMEM((4, ch, d), I32),
            pltpu.VMEM((4, ch, d), F32),
            [pltpu.SemaphoreType.DMA] * 8,
            [pltpu.SemaphoreType.DMA] * 8,
        ],
    )
    def gather_k(xpk_hbm, xdp_hbm, snd_hbm, rcv_hbm,
                 gsnd_hbm, grdp_hbm,
                 idx_s, idx_r, bsnd, brcv, sg, sw):
        wid = lax.axis_index("s") * nc + lax.axis_index("c")
        base = wid * per_w
        c1 = pltpu.async_copy(snd_hbm.at[wid], idx_s, sg[0])
        c2 = pltpu.async_copy(rcv_hbm.at[wid], idx_r, sg[1])
        c1.wait()
        c2.wait()

        def issue_gathers(j, b):
            pltpu.async_copy(xpk_hbm.at[idx_s.at[j]], bsnd.at[b], sg[b])
            pltpu.async_copy(xdp_hbm.at[idx_r.at[j]], brcv.at[b], sg[4 + b])

        def wait_gathers(j, b):
            pltpu.make_async_copy(
                xpk_hbm.at[idx_s.at[j]], bsnd.at[b], sg[b]).wait()
            pltpu.make_async_copy(
                xdp_hbm.at[idx_r.at[j]], brcv.at[b], sg[4 + b]).wait()

        def issue_writes(j, b):
            e0 = base + j * ch
            pltpu.async_copy(bsnd.at[b], gsnd_hbm.at[pl.ds(e0, ch)], sw[b])
            pltpu.async_copy(brcv.at[b], grdp_hbm.at[pl.ds(e0, ch)], sw[4 + b])

        def wait_writes(j, b):
            e0 = base + j * ch
            pltpu.make_async_copy(
                bsnd.at[b], gsnd_hbm.at[pl.ds(e0, ch)], sw[b]).wait()
            pltpu.make_async_copy(
                brcv.at[b], grdp_hbm.at[pl.ds(e0, ch)], sw[4 + b]).wait()

        issue_gathers(0, 0)
        issue_gathers(1, 1)
        nq = (nch - 2) // 4

        def outer(q, carry):
            for b in range(4):
                j = 4 * q + b
                wait_gathers(j, b)
                issue_writes(j, b)

                @pl.when(j >= 2)
                def _():
                    wait_writes(j - 2, (b + 2) % 4)

                issue_gathers(j + 2, (b + 2) % 4)
            return carry

        lax.fori_loop(0, nq, outer, 0)
        if nq > 0:
            wait_writes(4 * nq - 2, (4 * nq - 2) % 4)
            wait_writes(4 * nq - 1, (4 * nq - 1) % 4)
        for j in range(4 * nq, nch):
            wait_gathers(j, j % 4)
            issue_writes(j, j % 4)
            wait_writes(j, j % 4)
            if j + 2 < nch:
                issue_gathers(j + 2, (j + 2) % 4)

    return gather_k


def _mlp_body(ef_ref, ea_ref, gsnd_ref, grdp_ref,
              w1_ref, w2_ref, w3_ref, w4_ref, out_ref):
    w = gsnd_ref[...]
    gup = _unpack_lo(w)
    gds = _unpack_hi(w)[:, :64]
    aug = jnp.concatenate(
        [ef_ref[...], gds, grdp_ref[...][:, :64]], axis=1)
    h = _silu(_dot(aug, w1_ref[...]) * (1.0 / jnp.sqrt(136.0)))
    h = _silu(_dot(h, w2_ref[...]) * 0.0625)
    h = _silu(_dot(h, w3_ref[...]) * 0.0625)
    tpw = _dot(h, w4_ref[...]) * 0.0625
    ea = ea_ref[...]
    out_ref[0] = tpw[:, :128] * gup * ea[:, 0:1]
    wx = tpw[:, 128:] * gup
    out_ref[1] = wx * ea[:, 1:2]
    out_ref[2] = wx * ea[:, 2:3]
    out_ref[3] = wx * ea[:, 3:4]


def _edge_mlp(edge_feats, edge_attrs, gsnd, grdp, w1, w2, w3, w4, ec, d):
    be = 2000
    return pl.pallas_call(
        _mlp_body,
        grid=(ec // be,),
        in_specs=[
            pl.BlockSpec((be, 8), lambda i: (i, 0)),
            pl.BlockSpec((be, 4), lambda i: (i, 0)),
            pl.BlockSpec((be, d), lambda i: (i, 0)),
            pl.BlockSpec((be, d), lambda i: (i, 0)),
            pl.BlockSpec((136, 256), lambda i: (0, 0)),
            pl.BlockSpec((256, 256), lambda i: (0, 0)),
            pl.BlockSpec((256, 256), lambda i: (0, 0)),
            pl.BlockSpec((256, 256), lambda i: (0, 0)),
        ],
        out_specs=pl.BlockSpec((4, be, d), lambda i: (0, i, 0)),
        out_shape=jax.ShapeDtypeStruct((4, ec, d), F32),
    )(edge_feats, edge_attrs, gsnd, grdp, w1, w2, w3, w4)


def _build_scatter(ec, n, d):
    nc, ns = 2, 16
    per_t = ec // ns
    ch = SCH
    nch = per_t // ch
    nb = (n // ns) // 8 * 8
    tail = n - ns * nb

    mesh = plsc.VectorSubcoreMesh(
        core_axis_name="c", subcore_axis_name="s", num_cores=nc, num_subcores=ns)

    @functools.partial(
        pl.kernel,
        out_type=jax.ShapeDtypeStruct((4, n, d), F32),
        mesh=mesh,
        scratch_types=[
            pltpu.VMEM_SHARED((n, d), F32),
            pltpu.VMEM((nch, ch), jnp.int32),
            pltpu.VMEM((4, ch, d), F32),
            [pltpu.SemaphoreType.DMA] * 4,
            [pltpu.SemaphoreType.DMA] * 4,
        ],
    )
    def scatter_k(mji_hbm, rcv_hbm, zeros_hbm, msg_hbm,
                  acc, idxall, mbuf, sg, sa):
        c = lax.axis_index("c")
        s = lax.axis_index("s")
        pltpu.async_copy(rcv_hbm.at[s], idxall, sg[0]).wait()
        for p in range(2):
            g = c * 2 + p
            pltpu.sync_copy(zeros_hbm, acc.at[pl.ds(s * nb, nb)])

            @pl.when(s == ns - 1)
            def _zero_tail():
                pltpu.sync_copy(zeros_hbm.at[pl.ds(0, tail)],
                                acc.at[pl.ds(ns * nb, tail)])

            plsc.subcore_barrier()

            def issue_fetch(j, b):
                e0 = s * per_t + j * ch
                pltpu.async_copy(mji_hbm.at[g, pl.ds(e0, ch)],
                                 mbuf.at[b], sg[b])

            def wait_fetch(j, b):
                e0 = s * per_t + j * ch
                pltpu.make_async_copy(mji_hbm.at[g, pl.ds(e0, ch)],
                                      mbuf.at[b], sg[b]).wait()

            def issue_add(j, b):
                pltpu.async_copy(mbuf.at[b], acc.at[idxall.at[j]],
                                 sa[b], add=True)

            def wait_add(j, b):
                pltpu.make_async_copy(mbuf.at[b], acc.at[idxall.at[j]],
                                      sa[b]).wait()

            issue_fetch(0, 0)
            issue_fetch(1, 1)
            nq = (nch - 2) // 4

            def outer(q, carry):
                for b in range(4):
                    j = 4 * q + b
                    wait_fetch(j, b)
                    issue_add(j, b)

                    @pl.when(j >= 2)
                    def _():
                        wait_add(j - 2, (b + 2) % 4)

                    issue_fetch(j + 2, (b + 2) % 4)
                return carry

            lax.fori_loop(0, nq, outer, 0)
            if nq > 0:
                wait_add(4 * nq - 2, (4 * nq - 2) % 4)
                wait_add(4 * nq - 1, (4 * nq - 1) % 4)
            for j in range(4 * nq, nch):
                wait_fetch(j, j % 4)
                issue_add(j, j % 4)
                wait_add(j, j % 4)
                if j + 2 < nch:
                    issue_fetch(j + 2, (j + 2) % 4)
            plsc.subcore_barrier()
            pltpu.sync_copy(acc.at[pl.ds(s * nb, nb)],
                            msg_hbm.at[g, pl.ds(s * nb, nb)])

            @pl.when(s == ns - 1)
            def _write_tail():
                pltpu.sync_copy(acc.at[pl.ds(ns * nb, tail)],
                                msg_hbm.at[g, pl.ds(ns * nb, tail)])

            plsc.subcore_barrier()

    return scatter_k


def _out_body(*refs):
    msg_refs = refs[:NCHUNK]
    w0_ref, w1_ref, out_ref = refs[NCHUNK:]
    m = msg_refs[0][...]
    for r in msg_refs[1:]:
        m = m + r[...]
    scale = 1.0 / (jnp.sqrt(128.0) * 32.0)
    out_ref[0] = _dot(m[0], w0_ref[...]) * scale
    out_ref[1] = _dot(m[1], w1_ref[...]) * scale
    out_ref[2] = _dot(m[2], w1_ref[...]) * scale
    out_ref[3] = _dot(m[3], w1_ref[...]) * scale


def _out_linears(msgs, w_lin0, w_lin1, n, d):
    bn = 1000
    return pl.pallas_call(
        _out_body,
        grid=(n // bn,),
        in_specs=[pl.BlockSpec((4, bn, d), lambda i: (0, i, 0))
                  for _ in range(NCHUNK)] + [
            pl.BlockSpec((d, d), lambda i: (0, 0)),
            pl.BlockSpec((d, d), lambda i: (0, 0)),
        ],
        out_specs=pl.BlockSpec((4, bn, d), lambda i: (0, i, 0)),
        out_shape=jax.ShapeDtypeStruct((4, n, d), F32),
    )(*msgs, w_lin0, w_lin1)


def kernel(node_attrs, node_feats, edge_attrs, edge_feats, edge_index,
           W_up, W_down, W_skip, W_mlp1, W_mlp2, W_mlp3, W_mlp4,
           W_lin0, W_lin1):
    n, d = node_feats.shape
    e = edge_index.shape[0]
    dd = W_down.shape[1]
    ec = e // NCHUNK

    sender = edge_index[:, 0]
    receiver = edge_index[:, 1]
    w_cat = jnp.concatenate(
        [W_skip, W_up, W_down, jnp.zeros((d, d - dd), F32)], axis=1)

    sc, xpk, xdp = _node_linears(node_feats, w_cat, n, d)

    gather_k = _build_gather(ec, n, d)
    scatter_k = _build_scatter(ec, n, d)
    zeros = jnp.zeros(((n // 16) // 8 * 8, d), F32)

    gathered = []
    for i in range(NCHUNK):
        sl = slice(i * ec, (i + 1) * ec)
        snd3 = sender[sl].reshape(32, -1, GCH)
        rcv3g = receiver[sl].reshape(32, -1, GCH)
        gathered.append(gather_k(xpk, xdp, snd3, rcv3g))

    msgs = []
    for i in range(NCHUNK):
        sl = slice(i * ec, (i + 1) * ec)
        gsnd, grdp = gathered[i]
        mji = _edge_mlp(edge_feats[sl], edge_attrs[sl], gsnd, grdp,
                        W_mlp1, W_mlp2, W_mlp3, W_mlp4, ec, d)
        rcv3s = receiver[sl].reshape(16, -1, SCH)
        msgs.append(scatter_k(mji, rcv3s, zeros))

    out4 = _out_linears(msgs, W_lin0, W_lin1, n, d)
    reshaped = jnp.transpose(out4, (1, 2, 0))
    return (reshaped, sc)
